# baseline (speedup 1.0000x reference)
import functools

import jax
import jax.numpy as jnp
import numpy as np
from jax import lax
from jax.experimental import pallas as pl
from jax.experimental.pallas import tpu as pltpu
from jax.experimental.pallas import tpu_sc as plsc

B, Q, C, G, P = 64, 100, 3, 30, 20
ALPHA, GAMMA = 0.25, 2.0
NUM_BOXES = float(B * G)

NW = 32
NR = G * P * 2


def _log1p01(e):
    u = e / (2.0 + e)
    w = u * u
    poly = 1.0 + w * (1.0 / 3.0 + w * (1.0 / 5.0 + w * (1.0 / 7.0 + w * (1.0 / 9.0))))
    return 2.0 * u * poly


def _sc_body(lg_hbm, lb_hbm, out_hbm, lg2_s, lab_s, out_s, sem):
    cc = lax.axis_index("c")
    ss = lax.axis_index("s")
    wid = ss * 2 + cc
    b0 = wid * 2
    lo = b0 & 7

    h_lg2 = [pltpu.async_copy(lg_hbm.at[pl.ds(c * B + 8 * (wid >> 2), 8)],
                              lg2_s.at[pl.ds(8 * c, 8)], sem)
             for c in range(C)]
    h_lab = pltpu.async_copy(lb_hbm, lab_s, sem)

    iota16 = lax.iota(jnp.int32, 16)
    zero16 = jnp.zeros((16,), jnp.float32)

    for h in h_lg2:
        h.wait()
    h_lab.wait()

    def t2_body(i, acc):
        qg = i * 16 + iota16
        qgc = jnp.minimum(qg, 2 * G - 1)
        pi = qgc // G
        qq = qgc - pi * G
        lab = plsc.load_gather(lab_s, [qq, b0 + pi])
        x = plsc.load_gather(lg2_s, [lab * 8 + lo + pi, qq])
        e = jnp.exp(-jnp.abs(x))
        inv = 1.0 / (1.0 + e)
        p = jnp.where(x >= 0.0, inv, 1.0 - inv)
        ce0 = jnp.maximum(x, 0.0) + _log1p01(e)
        f0 = (1.0 - ALPHA) * (p * p) * ce0
        omp = 1.0 - p
        f1 = ALPHA * (omp * omp) * (ce0 - x)
        return acc + jnp.where(qg < 2 * G, f1 - f0, 0.0)

    acc_ce = lax.fori_loop(0, 4, t2_body, zero16)

    out16 = jnp.where(iota16 == 0, jnp.sum(acc_ce), 0.0) / NUM_BOXES
    out_s[...] = out16
    pltpu.sync_copy(out_s, out_hbm.at[wid])


@functools.cache
def _sc_call():
    return pl.kernel(
        _sc_body,
        out_type=jax.ShapeDtypeStruct((NW, 16), jnp.float32),
        mesh=plsc.VectorSubcoreMesh(core_axis_name="c", subcore_axis_name="s",
                                    num_cores=2, num_subcores=16),
        compiler_params=pltpu.CompilerParams(needs_layout_passes=False),
        scratch_types=[
            pltpu.VMEM((3 * 8, Q), jnp.float32),
            pltpu.VMEM((G, B), jnp.int32),
            pltpu.VMEM((16,), jnp.float32),
            pltpu.SemaphoreType.DMA,
        ],
    )


def _tc_body(lg_ref, pp_ref, tg_ref, out_ref):
    x = lg_ref[...]
    e = jnp.exp(-jnp.abs(x))
    inv = 1.0 / (1.0 + e)
    p = jnp.where(x >= 0.0, inv, 1.0 - inv)
    ce0 = jnp.maximum(x, 0.0) + jnp.log1p(e)
    t1 = jnp.sum((1.0 - ALPHA) * (p * p) * ce0)

    tgm = jnp.transpose(
        jnp.transpose(tg_ref[...], (1, 0)).reshape(B, G, 2 * P), (0, 2, 1))
    s3 = pp_ref[...].reshape(B, 2 * P, Q)[:, :, :G]
    l1 = jnp.sum(jnp.abs(s3 - tgm))

    sd = s3[:, 2:2 * P, :] - s3[:, 0:2 * P - 2, :]
    td = tgm[:, 2:2 * P, :] - tgm[:, 0:2 * P - 2, :]
    qd = sd * td
    qs = sd * sd
    qt = td * td
    ud = qd[:, 0:37, :] + qd[:, 1:38, :]
    us = qs[:, 0:37, :] + qs[:, 1:38, :]
    ut = qt[:, 0:37, :] + qt[:, 1:38, :]
    cos = ud * lax.rsqrt(jnp.maximum(us * ut, 1e-24))
    msk = lax.broadcasted_iota(jnp.int32, (B, 37, G), 1) % 2 == 0
    dirv = jnp.sum(jnp.where(msk, 1.0 - cos, 0.0))

    col = lax.broadcasted_iota(jnp.int32, (8, 128), 1)
    out_ref[...] = jnp.where(col == 0, t1,
                             jnp.where(col == 1, l1,
                                       jnp.where(col == 2, dirv,
                                                 0.0))) / NUM_BOXES


@functools.cache
def _tc_call():
    return pl.pallas_call(
        _tc_body,
        out_shape=jax.ShapeDtypeStruct((8, 128), jnp.float32),
    )


def kernel(pred_logits, pred_points, labels, target_points):
    lg = jnp.transpose(pred_logits, (2, 0, 1)).reshape(C * B, Q)
    pp = jnp.transpose(pred_points, (0, 2, 3, 1)).reshape(B * P * 2, Q)
    lb = jnp.transpose(labels, (1, 0)).astype(jnp.int32)
    tg = jnp.transpose(target_points, (1, 2, 3, 0)).reshape(NR, B)
    sc_part = _sc_call()(lg, lb)
    tc_part = _tc_call()(lg, pp, tg)
    return sc_part.sum(axis=0)[:3] + tc_part[0, :3]


# speedup vs baseline: 1.1193x; 1.1193x over previous
import functools

import jax
import jax.numpy as jnp
import numpy as np
from jax import lax
from jax.experimental import pallas as pl
from jax.experimental.pallas import tpu as pltpu
from jax.experimental.pallas import tpu_sc as plsc

B, Q, C, G, P = 64, 100, 3, 30, 20
ALPHA, GAMMA = 0.25, 2.0
NUM_BOXES = float(B * G)

NW = 32
NR = G * P * 2


def _log1p01(e):
    u = e / (2.0 + e)
    w = u * u
    poly = 1.0 + w * (1.0 / 3.0 + w * (1.0 / 5.0 + w * (1.0 / 7.0 + w * (1.0 / 9.0))))
    return 2.0 * u * poly


def _sc_body(lg_hbm, lb_hbm, out_hbm, lg2_s, lab_s, out_s, sem):
    wid = lax.axis_index("s")
    b0 = wid * 4
    lo = b0 & 7

    h_lg2 = [pltpu.async_copy(lg_hbm.at[pl.ds(c * B + 8 * (wid >> 1), 8)],
                              lg2_s.at[pl.ds(8 * c, 8)], sem)
             for c in range(C)]
    h_lab = pltpu.async_copy(lb_hbm, lab_s, sem)

    iota16 = lax.iota(jnp.int32, 16)
    zero16 = jnp.zeros((16,), jnp.float32)

    for h in h_lg2:
        h.wait()
    h_lab.wait()

    def t2_body(i, acc):
        qg = i * 16 + iota16
        qgc = jnp.minimum(qg, 4 * G - 1)
        pi = qgc // G
        qq = qgc - pi * G
        lab = plsc.load_gather(lab_s, [qq, b0 + pi])
        x = plsc.load_gather(lg2_s, [lab * 8 + lo + pi, qq])
        e = jnp.exp(-jnp.abs(x))
        inv = 1.0 / (1.0 + e)
        p = jnp.where(x >= 0.0, inv, 1.0 - inv)
        ce0 = jnp.maximum(x, 0.0) + _log1p01(e)
        f0 = (1.0 - ALPHA) * (p * p) * ce0
        omp = 1.0 - p
        f1 = ALPHA * (omp * omp) * (ce0 - x)
        return acc + jnp.where(qg < 4 * G, f1 - f0, 0.0)

    acc_ce = lax.fori_loop(0, 8, t2_body, zero16)

    out16 = jnp.where(iota16 == 0, jnp.sum(acc_ce), 0.0) / NUM_BOXES
    out_s[...] = out16
    pltpu.sync_copy(out_s, out_hbm.at[wid])


@functools.cache
def _sc_call():
    return pl.kernel(
        _sc_body,
        out_type=jax.ShapeDtypeStruct((16, 16), jnp.float32),
        mesh=plsc.VectorSubcoreMesh(core_axis_name="c", subcore_axis_name="s",
                                    num_cores=1, num_subcores=16),
        compiler_params=pltpu.CompilerParams(needs_layout_passes=False),
        scratch_types=[
            pltpu.VMEM((3 * 8, Q), jnp.float32),
            pltpu.VMEM((G, B), jnp.int32),
            pltpu.VMEM((16,), jnp.float32),
            pltpu.SemaphoreType.DMA,
        ],
    )


def _tc_body(lg_ref, pp_ref, tg_ref, out_ref):
    x = lg_ref[...]
    e = jnp.exp(-jnp.abs(x))
    inv = 1.0 / (1.0 + e)
    p = jnp.where(x >= 0.0, inv, 1.0 - inv)
    ce0 = jnp.maximum(x, 0.0) + jnp.log1p(e)
    t1 = jnp.sum((1.0 - ALPHA) * (p * p) * ce0)

    tgm = jnp.transpose(
        jnp.transpose(tg_ref[...], (1, 0)).reshape(B, G, 2 * P), (0, 2, 1))
    s3 = pp_ref[...].reshape(B, 2 * P, Q)[:, :, :G]
    l1 = jnp.sum(jnp.abs(s3 - tgm))

    sd = s3[:, 2:2 * P, :] - s3[:, 0:2 * P - 2, :]
    td = tgm[:, 2:2 * P, :] - tgm[:, 0:2 * P - 2, :]
    qd = sd * td
    qs = sd * sd
    qt = td * td
    ud = qd[:, 0:37, :] + qd[:, 1:38, :]
    us = qs[:, 0:37, :] + qs[:, 1:38, :]
    ut = qt[:, 0:37, :] + qt[:, 1:38, :]
    cos = ud * lax.rsqrt(jnp.maximum(us * ut, 1e-24))
    msk = lax.broadcasted_iota(jnp.int32, (B, 37, G), 1) % 2 == 0
    dirv = jnp.sum(jnp.where(msk, 1.0 - cos, 0.0))

    col = lax.broadcasted_iota(jnp.int32, (8, 128), 1)
    out_ref[...] = jnp.where(col == 0, t1,
                             jnp.where(col == 1, l1,
                                       jnp.where(col == 2, dirv,
                                                 0.0))) / NUM_BOXES


@functools.cache
def _tc_call():
    return pl.pallas_call(
        _tc_body,
        out_shape=jax.ShapeDtypeStruct((8, 128), jnp.float32),
    )


def kernel(pred_logits, pred_points, labels, target_points):
    lg = jnp.transpose(pred_logits, (2, 0, 1)).reshape(C * B, Q)
    pp = jnp.transpose(pred_points, (0, 2, 3, 1)).reshape(B * P * 2, Q)
    lb = jnp.transpose(labels, (1, 0)).astype(jnp.int32)
    tg = jnp.transpose(target_points, (1, 2, 3, 0)).reshape(NR, B)
    sc_part = _sc_call()(lg, lb)
    tc_part = _tc_call()(lg, pp, tg)
    return sc_part.sum(axis=0)[:3] + tc_part[0, :3]
